# baseline (device time: 588186 ns/iter reference)
import jax
import jax.numpy as jnp
from jax import lax
from jax.experimental import pallas as pl
from jax.experimental.pallas import tpu as pltpu

N_DEV = 4


def kernel(x, w_mat, scale_x, scale_w):
    m_glob, k_loc = x.shape
    _, n = w_mat.shape
    m_per = m_glob // N_DEV
    tn = 512
    nh = (n // 2) // tn
    n_hops = N_DEV - 1

    def body(x_ref, w_ref, sx_ref, sw_ref, out_ref,
             s_cw_ref, r_cw_ref, s_ccw_ref, r_ccw_ref,
             acc_ref, r_ref, xb_ref, wb_ref,
             send_cw, recv_cw, send_ccw, recv_ccw,
             copy_sem):
        my = lax.axis_index("i")
        left = lax.rem(my + N_DEV - 1, N_DEV)
        right = lax.rem(my + 1, N_DEV)

        barrier = pltpu.get_barrier_semaphore()
        for nbr in (left, right):
            pl.semaphore_signal(barrier, inc=1, device_id=(nbr,),
                                device_id_type=pl.DeviceIdType.MESH)
        pl.semaphore_wait(barrier, 2)

        scale = sx_ref[0] * sw_ref[0]

        xb_ref[...] = x_ref[...].astype(jnp.bfloat16)
        wb_ref[...] = w_ref[...].astype(jnp.bfloat16)

        def copy(src, dst):
            c = pltpu.make_async_copy(src, dst, copy_sem)
            c.start()
            c.wait()

        dirs = [
            dict(s=s_cw_ref, r=r_cw_ref, ssem=send_cw, rsem=recv_cw,
                 dst=right, col0=0),
            dict(s=s_ccw_ref, r=r_ccw_ref, ssem=send_ccw, rsem=recv_ccw,
                 dst=left, col0=nh * tn),
        ]

        def desc(di, h, j):
            d = dirs[di]
            return pltpu.make_async_remote_copy(
                src_ref=d["s"].at[j], dst_ref=d["r"].at[h, j],
                send_sem=d["ssem"].at[h, j], recv_sem=d["rsem"].at[h, j],
                device_id=(d["dst"],), device_id_type=pl.DeviceIdType.MESH,
            )

        def chunk_idx(di, h):
            if di == 0:
                return lax.rem(my + N_DEV - 2 - h, N_DEV)
            return lax.rem(my + 2 + h, N_DEV)

        def x_tile(c_idx):
            return xb_ref[pl.ds(c_idx * m_per, m_per), :]

        xts = [x_tile(chunk_idx(0, -1)), x_tile(chunk_idx(1, -1))]
        for j in range(nh):
            for di in (0, 1):
                d = dirs[di]
                c0 = d["col0"] + j * tn
                acc = lax.dot_general(
                    xts[di], wb_ref[:, c0:c0 + tn], (((1,), (0,)), ((), ())),
                    preferred_element_type=jnp.float32,
                )
                acc_ref[...] = acc
                copy(acc_ref, d["s"].at[j])
                desc(di, 0, j).start()

        for h in range(n_hops):
            last = h == n_hops - 1
            xts = [x_tile(chunk_idx(0, h)), x_tile(chunk_idx(1, h))]
            for j in range(nh):
                for di in (0, 1):
                    d = dirs[di]
                    c0 = d["col0"] + j * tn
                    desc(di, h, j).wait()
                    rc = pltpu.make_async_copy(d["r"].at[h, j], r_ref,
                                               copy_sem)
                    rc.start()
                    acc = lax.dot_general(
                        xts[di], wb_ref[:, c0:c0 + tn],
                        (((1,), (0,)), ((), ())),
                        preferred_element_type=jnp.float32,
                    )
                    rc.wait()
                    acc = acc + r_ref[...]
                    if last:
                        acc_ref[...] = jnp.maximum(acc * scale, 0.0)
                        copy(acc_ref, out_ref.at[:, pl.ds(c0, tn)])
                    else:
                        acc_ref[...] = acc
                        copy(acc_ref, d["s"].at[j])
                        desc(di, h + 1, j).start()

    s_shape = jax.ShapeDtypeStruct((nh, m_per, tn), jnp.float32)
    r_shape = jax.ShapeDtypeStruct((n_hops, nh, m_per, tn), jnp.float32)
    out = pl.pallas_call(
        body,
        out_shape=[
            jax.ShapeDtypeStruct((m_per, n), jnp.float32),
            s_shape, r_shape,
            s_shape, r_shape,
        ],
        in_specs=[
            pl.BlockSpec(memory_space=pltpu.VMEM),
            pl.BlockSpec(memory_space=pltpu.VMEM),
            pl.BlockSpec(memory_space=pltpu.SMEM),
            pl.BlockSpec(memory_space=pltpu.SMEM),
        ],
        out_specs=[
            pl.BlockSpec(memory_space=pl.ANY),
            pl.BlockSpec(memory_space=pltpu.MemorySpace.HBM),
            pl.BlockSpec(memory_space=pltpu.MemorySpace.HBM),
            pl.BlockSpec(memory_space=pltpu.MemorySpace.HBM),
            pl.BlockSpec(memory_space=pltpu.MemorySpace.HBM),
        ],
        scratch_shapes=[
            pltpu.VMEM((m_per, tn), jnp.float32),
            pltpu.VMEM((m_per, tn), jnp.float32),
            pltpu.VMEM((m_glob, k_loc), jnp.bfloat16),
            pltpu.VMEM((k_loc, n), jnp.bfloat16),
            pltpu.SemaphoreType.DMA((n_hops, nh)),
            pltpu.SemaphoreType.DMA((n_hops, nh)),
            pltpu.SemaphoreType.DMA((n_hops, nh)),
            pltpu.SemaphoreType.DMA((n_hops, nh)),
            pltpu.SemaphoreType.DMA,
        ],
        compiler_params=pltpu.CompilerParams(collective_id=0),
    )(x, w_mat, scale_x, scale_w)[0]
    return out


# device time: 585013 ns/iter; 1.0054x vs baseline; 1.0054x over previous
import jax
import jax.numpy as jnp
from jax import lax
from jax.experimental import pallas as pl
from jax.experimental.pallas import tpu as pltpu

N_DEV = 4


def kernel(x, w_mat, scale_x, scale_w):
    m_glob, k_loc = x.shape
    _, n = w_mat.shape
    m_per = m_glob // N_DEV
    tn = 512
    nh = (n // 2) // tn
    n_hops = N_DEV - 1

    def body(x_ref, w_ref, sx_ref, sw_ref, out_ref,
             s_cw_ref, r_cw_ref, s_ccw_ref, r_ccw_ref,
             acc_ref, r_ref, send_cw, recv_cw, send_ccw, recv_ccw,
             copy_sem):
        my = lax.axis_index("i")
        left = lax.rem(my + N_DEV - 1, N_DEV)
        right = lax.rem(my + 1, N_DEV)

        barrier = pltpu.get_barrier_semaphore()
        for nbr in (left, right):
            pl.semaphore_signal(barrier, inc=1, device_id=(nbr,),
                                device_id_type=pl.DeviceIdType.MESH)
        pl.semaphore_wait(barrier, 2)

        scale = sx_ref[0] * sw_ref[0]

        def copy(src, dst):
            c = pltpu.make_async_copy(src, dst, copy_sem)
            c.start()
            c.wait()

        dirs = [
            dict(s=s_cw_ref, r=r_cw_ref, ssem=send_cw, rsem=recv_cw,
                 dst=right, col0=0),
            dict(s=s_ccw_ref, r=r_ccw_ref, ssem=send_ccw, rsem=recv_ccw,
                 dst=left, col0=nh * tn),
        ]

        def desc(di, h, j):
            d = dirs[di]
            return pltpu.make_async_remote_copy(
                src_ref=d["s"].at[j], dst_ref=d["r"].at[h, j],
                send_sem=d["ssem"].at[h, j], recv_sem=d["rsem"].at[h, j],
                device_id=(d["dst"],), device_id_type=pl.DeviceIdType.MESH,
            )

        def chunk_idx(di, h):
            if di == 0:
                return lax.rem(my + N_DEV - 2 - h, N_DEV)
            return lax.rem(my + 2 + h, N_DEV)

        def x_tile(c_idx):
            return x_ref[pl.ds(c_idx * m_per, m_per), :].astype(jnp.bfloat16)

        xts = [x_tile(chunk_idx(0, -1)), x_tile(chunk_idx(1, -1))]
        for j in range(nh):
            for di in (0, 1):
                d = dirs[di]
                c0 = d["col0"] + j * tn
                wt = w_ref[:, c0:c0 + tn].astype(jnp.bfloat16)
                acc = lax.dot_general(
                    xts[di], wt, (((1,), (0,)), ((), ())),
                    preferred_element_type=jnp.float32,
                )
                acc_ref[...] = acc
                copy(acc_ref, d["s"].at[j])
                desc(di, 0, j).start()

        for h in range(n_hops):
            last = h == n_hops - 1
            xts = [x_tile(chunk_idx(0, h)), x_tile(chunk_idx(1, h))]
            for j in range(nh):
                for di in (0, 1):
                    d = dirs[di]
                    c0 = d["col0"] + j * tn
                    desc(di, h, j).wait()
                    wt = w_ref[:, c0:c0 + tn].astype(jnp.bfloat16)
                    acc = lax.dot_general(
                        xts[di], wt, (((1,), (0,)), ((), ())),
                        preferred_element_type=jnp.float32,
                    )
                    copy(d["r"].at[h, j], r_ref)
                    acc = acc + r_ref[...]
                    if last:
                        acc_ref[...] = jnp.maximum(acc * scale, 0.0)
                        copy(acc_ref, out_ref.at[:, pl.ds(c0, tn)])
                    else:
                        acc_ref[...] = acc
                        copy(acc_ref, d["s"].at[j])
                        desc(di, h + 1, j).start()

    s_shape = jax.ShapeDtypeStruct((nh, m_per, tn), jnp.float32)
    r_shape = jax.ShapeDtypeStruct((n_hops, nh, m_per, tn), jnp.float32)
    out = pl.pallas_call(
        body,
        out_shape=[
            jax.ShapeDtypeStruct((m_per, n), jnp.float32),
            s_shape, r_shape,
            s_shape, r_shape,
        ],
        in_specs=[
            pl.BlockSpec(memory_space=pltpu.VMEM),
            pl.BlockSpec(memory_space=pltpu.VMEM),
            pl.BlockSpec(memory_space=pltpu.SMEM),
            pl.BlockSpec(memory_space=pltpu.SMEM),
        ],
        out_specs=[
            pl.BlockSpec(memory_space=pl.ANY),
            pl.BlockSpec(memory_space=pltpu.MemorySpace.HBM),
            pl.BlockSpec(memory_space=pltpu.MemorySpace.HBM),
            pl.BlockSpec(memory_space=pltpu.MemorySpace.HBM),
            pl.BlockSpec(memory_space=pltpu.MemorySpace.HBM),
        ],
        scratch_shapes=[
            pltpu.VMEM((m_per, tn), jnp.float32),
            pltpu.VMEM((m_per, tn), jnp.float32),
            pltpu.SemaphoreType.DMA((n_hops, nh)),
            pltpu.SemaphoreType.DMA((n_hops, nh)),
            pltpu.SemaphoreType.DMA((n_hops, nh)),
            pltpu.SemaphoreType.DMA((n_hops, nh)),
            pltpu.SemaphoreType.DMA,
        ],
        compiler_params=pltpu.CompilerParams(collective_id=0),
    )(x, w_mat, scale_x, scale_w)[0]
    return out
